# baseline (speedup 1.0000x reference)
import numpy as np
import jax
import jax.numpy as jnp
from jax.experimental import pallas as pl
from jax.experimental.pallas import tpu as pltpu

_NUM_CLASSES = 80
_CONF = 0.05
_IOU = 0.5
_MPC = 100
_MAXDET = 100
_CAND = 1000
_N = 49104
_NT = 49152
_NB = 1024
_CH = 16
_W = _NT // _CH
_BPC = _W // _NB
_NEG = float("-inf")
_PADV = -1.0e30


def _anchor_table(image_h, image_w):
    aspect_ratios = [0.5, 1.0, 2.0]
    scales = [2 ** x for x in [0.0, 1.0 / 3.0, 2.0 / 3.0]]
    areas = [x ** 2 for x in [32.0, 64.0, 128.0, 256.0, 512.0]]
    all_anchors = []
    for level in range(3, 8):
        stride = 2 ** level
        area = areas[level - 3]
        dims = []
        for ratio in aspect_ratios:
            ah = np.sqrt(area / ratio)
            aw = area / ah
            for s in scales:
                dims.append([aw * s, ah * s])
        dims = np.asarray(dims, dtype=np.float32)
        fh = int(np.ceil(image_h / stride))
        fw = int(np.ceil(image_w / stride))
        rx = (np.arange(fw, dtype=np.float32) + 0.5) * stride
        ry = (np.arange(fh, dtype=np.float32) + 0.5) * stride
        cx, cy = np.meshgrid(rx, ry)
        centers = np.stack([cx, cy], axis=-1).reshape(-1, 1, 2)
        centers = np.tile(centers, (1, 9, 1))
        dims_t = np.tile(dims[None, :, :], (centers.shape[0], 1, 1))
        anchors = np.concatenate([centers, dims_t], axis=-1).reshape(-1, 4)
        all_anchors.append(anchors)
    return np.concatenate(all_anchors, axis=0).astype(np.float32)


def _prep_body(pred_ref, anch_ref, scores_ref, boxes_ref):
    i = pl.program_id(0)
    s = jax.nn.sigmoid(pred_ref[:, 4:84, :])
    col = jax.lax.broadcasted_iota(jnp.int32, s.shape, 2) + i * _NB
    scores_ref[0] = jnp.where(col < _N, s, -1.0)

    acx = anch_ref[:, 0:1, :]
    acy = anch_ref[:, 1:2, :]
    aw = anch_ref[:, 2:3, :]
    ah = anch_ref[:, 3:4, :]
    b0 = pred_ref[:, 0:1, :] * 0.1
    b1 = pred_ref[:, 1:2, :] * 0.1
    b2 = pred_ref[:, 2:3, :] * 0.2
    b3 = pred_ref[:, 3:4, :] * 0.2
    cx = b0 * aw + acx
    cy = b1 * ah + acy
    w = jnp.exp(b2) * aw
    h = jnp.exp(b3) * ah
    x1 = cx - w / 2.0
    y1 = cy - h / 2.0
    x2 = cx + w / 2.0
    y2 = cy + h / 2.0
    area = jnp.maximum(x2 - x1, 0.0) * jnp.maximum(y2 - y1, 0.0)
    zpad = jnp.zeros_like(area)
    boxes_ref[0] = jnp.concatenate(
        [x1, y1, x2, y2, area, zpad, zpad, zpad], axis=1)


def _nms_body(scores_ref, boxes_ref, fo_ref, pcs_ref, px1_ref, py1_ref,
              px2_ref, py2_ref):
    B = scores_ref.shape[1]

    def _civ(c):
        iv = jax.lax.broadcasted_iota(jnp.int32, (B, _NUM_CLASSES, _W), 2)
        return iv + c * _W

    def _count_ge(trial_f):
        def chunk(c, cnt):
            sc = scores_ref[c]
            return cnt + jnp.sum((sc >= trial_f).astype(jnp.int32), axis=2,
                                 keepdims=True)
        return jax.lax.fori_loop(0, _CH, chunk,
                                 jnp.zeros((B, _NUM_CLASSES, 1), jnp.int32))

    def bit_step(j, t):
        trial = t | jax.lax.shift_left(1, 30 - j)
        trial_f = jax.lax.bitcast_convert_type(trial, jnp.float32)
        return jnp.where(_count_ge(trial_f) >= _CAND, trial, t)

    t = jax.lax.fori_loop(0, 31, bit_step, jnp.zeros((B, 80, 1), jnp.int32))
    t_f = jax.lax.bitcast_convert_type(t, jnp.float32)

    def chunk_strict(c, cnt):
        sc = scores_ref[c]
        return cnt + jnp.sum((sc > t_f).astype(jnp.int32), axis=2,
                             keepdims=True)

    n_strict = jax.lax.fori_loop(0, _CH, chunk_strict,
                                 jnp.zeros((B, _NUM_CLASSES, 1), jnp.int32))
    quota = _CAND - n_strict

    def idx_step(j, cc):
        trial = cc | jax.lax.shift_left(1, 16 - j)

        def chunk(c, cnt):
            sc = scores_ref[c]
            hit = (sc == t_f) & (_civ(c) <= trial)
            return cnt + jnp.sum(hit.astype(jnp.int32), axis=2, keepdims=True)

        cnt = jax.lax.fori_loop(0, _CH, chunk,
                                jnp.zeros((B, _NUM_CLASSES, 1), jnp.int32))
        return jnp.where(cnt <= quota, trial, cc)

    cut = jax.lax.fori_loop(0, 17, idx_step, jnp.zeros((B, 80, 1), jnp.int32))

    def chunk_keep(c, mp):
        m, pick = mp
        sc = scores_ref[c]
        keep = (sc > t_f) | ((sc == t_f) & (_civ(c) <= cut))
        sc = jnp.where(keep, sc, _NEG)
        scores_ref[c] = sc
        cm = jnp.max(sc, axis=2, keepdims=True)
        ci = jnp.min(jnp.where(sc == cm, _civ(c), _NT), axis=2, keepdims=True)
        better = cm > m
        return (jnp.maximum(m, cm), jnp.where(better, ci, pick))

    m0, pick0 = jax.lax.fori_loop(
        0, _CH, chunk_keep,
        (jnp.full((B, _NUM_CLASSES, 1), _NEG, jnp.float32),
         jnp.full((B, _NUM_CLASSES, 1), _NT, jnp.int32)))

    pcs_ref[...] = jnp.full(pcs_ref.shape, _PADV, jnp.float32)
    px1_ref[...] = jnp.zeros(px1_ref.shape, jnp.float32)
    py1_ref[...] = jnp.zeros(py1_ref.shape, jnp.float32)
    px2_ref[...] = jnp.zeros(px2_ref.shape, jnp.float32)
    py2_ref[...] = jnp.zeros(py2_ref.shape, jnp.float32)

    slot = jax.lax.broadcasted_iota(jnp.int32, (B, _NUM_CLASSES, 128), 2)

    def nms_step(k, mp):
        m, pick = mp

        def p2(c, acc):
            oh = _civ(c) == pick
            bx = boxes_ref[c]
            out = []
            for r in range(4):
                g = jnp.max(jnp.where(oh, bx[:, r:r + 1, :], _PADV), axis=2,
                            keepdims=True)
                out.append(jnp.maximum(acc[r], g))
            return tuple(out)

        init = tuple(jnp.full((B, _NUM_CLASSES, 1), _PADV, jnp.float32)
                     for _ in range(4))
        p_x1, p_y1, p_x2, p_y2 = jax.lax.fori_loop(0, _CH, p2, init)

        a1 = (jnp.maximum(p_x2 - p_x1, 0.0) *
              jnp.maximum(p_y2 - p_y1, 0.0))

        def p3(c, mp2):
            m2, pick2 = mp2
            sc = scores_ref[c]
            bx = boxes_ref[c]
            ix1 = jnp.maximum(p_x1, bx[:, 0:1, :])
            iy1 = jnp.maximum(p_y1, bx[:, 1:2, :])
            ix2 = jnp.minimum(p_x2, bx[:, 2:3, :])
            iy2 = jnp.minimum(p_y2, bx[:, 3:4, :])
            inter = jnp.maximum(ix2 - ix1, 0.0) * jnp.maximum(iy2 - iy1, 0.0)
            iou = inter / jnp.maximum(a1 + bx[:, 4:5, :] - inter, 1e-8)
            kill = (iou > _IOU) | (_civ(c) == pick)
            sc = jnp.where(kill, _NEG, sc)
            scores_ref[c] = sc
            cm = jnp.max(sc, axis=2, keepdims=True)
            ci = jnp.min(jnp.where(sc == cm, _civ(c), _NT), axis=2,
                         keepdims=True)
            better = cm > m2
            return (jnp.maximum(m2, cm), jnp.where(better, ci, pick2))

        m_next, pick_next = jax.lax.fori_loop(
            0, _CH, p3,
            (jnp.full((B, _NUM_CLASSES, 1), _NEG, jnp.float32),
             jnp.full((B, _NUM_CLASSES, 1), _NT, jnp.int32)))

        valid = m >= _CONF
        hit = slot == k
        pcs_ref[...] = jnp.where(hit, jnp.where(valid, m, -1.0), pcs_ref[...])
        px1_ref[...] = jnp.where(hit, jnp.where(valid, p_x1, 0.0), px1_ref[...])
        py1_ref[...] = jnp.where(hit, jnp.where(valid, p_y1, 0.0), py1_ref[...])
        px2_ref[...] = jnp.where(hit, jnp.where(valid, p_x2, 0.0), px2_ref[...])
        py2_ref[...] = jnp.where(hit, jnp.where(valid, p_y2, 0.0), py2_ref[...])
        return (m_next, pick_next)

    jax.lax.fori_loop(0, _MPC, nms_step, (m0, pick0))

    ci = jax.lax.broadcasted_iota(jnp.int32, (B, _NUM_CLASSES, 128), 1)
    flatv = ci * 128 + slot
    orow = jax.lax.broadcasted_iota(jnp.int32, (B, 128, 8), 1)

    fo_ref[...] = jnp.zeros(fo_ref.shape, jnp.float32)

    def merge_step(k, nv):
        fs = pcs_ref[...]
        m2 = jnp.max(jnp.max(fs, axis=2, keepdims=True), axis=1, keepdims=True)
        pf = jnp.where(fs == m2, flatv, 2 ** 30)
        pf = jnp.min(jnp.min(pf, axis=2, keepdims=True), axis=1, keepdims=True)
        onehot = flatv == pf
        pcs_ref[...] = jnp.where(onehot, _PADV, fs)

        def gath(r):
            g = jnp.where(onehot, r[...], _PADV)
            return jnp.max(jnp.max(g, axis=2, keepdims=True), axis=1,
                           keepdims=True)

        v = m2 > -0.5
        nv = nv + v.astype(jnp.float32)
        ws = jnp.where(v, m2, 0.0)
        wx1 = jnp.where(v, gath(px1_ref), 0.0)
        wy1 = jnp.where(v, gath(py1_ref), 0.0)
        wx2 = jnp.where(v, gath(px2_ref), 0.0)
        wy2 = jnp.where(v, gath(py2_ref), 0.0)
        wc = jnp.where(v, (pf // 128).astype(jnp.float32), 0.0)
        row = jnp.concatenate([ws, wx1, wy1, wx2, wy2, wc, nv,
                               jnp.zeros_like(ws)], axis=2)
        fo_ref[...] = jnp.where(orow == k, row, fo_ref[...])
        return nv

    jax.lax.fori_loop(0, _MAXDET, merge_step,
                      jnp.zeros((B, 1, 1), jnp.float32))


def kernel(images, predictions):
    B, n, _ = predictions.shape
    del images
    anch = _anchor_table(512.0, 512.0)
    anch = np.pad(anch, ((0, _NT - _N), (0, 4)),
                  constant_values=1.0)
    anch_t = jnp.asarray(anch.T[None], jnp.float32)

    pred = jnp.pad(predictions, ((0, 0), (0, _NT - n), (0, 0)))
    pred_t = jnp.transpose(pred, (0, 2, 1))

    scores, boxes = pl.pallas_call(
        _prep_body,
        grid=(_NT // _NB,),
        in_specs=[
            pl.BlockSpec((B, 84, _NB), lambda i: (0, 0, i)),
            pl.BlockSpec((1, 8, _NB), lambda i: (0, 0, i)),
        ],
        out_specs=[
            pl.BlockSpec((1, B, 80, _NB),
                         lambda i: (i // _BPC, 0, 0, i % _BPC)),
            pl.BlockSpec((1, B, 8, _NB),
                         lambda i: (i // _BPC, 0, 0, i % _BPC)),
        ],
        out_shape=[
            jax.ShapeDtypeStruct((_CH, B, 80, _W), jnp.float32),
            jax.ShapeDtypeStruct((_CH, B, 8, _W), jnp.float32),
        ],
    )(pred_t, anch_t)

    fo = pl.pallas_call(
        _nms_body,
        out_shape=jax.ShapeDtypeStruct((B, 128, 8), jnp.float32),
        scratch_shapes=[
            pltpu.VMEM((B, _NUM_CLASSES, 128), jnp.float32),
            pltpu.VMEM((B, _NUM_CLASSES, 128), jnp.float32),
            pltpu.VMEM((B, _NUM_CLASSES, 128), jnp.float32),
            pltpu.VMEM((B, _NUM_CLASSES, 128), jnp.float32),
            pltpu.VMEM((B, _NUM_CLASSES, 128), jnp.float32),
        ],
    )(scores, boxes)

    out_scores = fo[:, :_MAXDET, 0]
    out_boxes = fo[:, :_MAXDET, 1:5]
    out_classes = fo[:, :_MAXDET, 5]
    n_valid = fo[:, _MAXDET - 1, 6].astype(jnp.int32)
    return out_boxes, out_scores, out_classes, n_valid


# speedup vs baseline: 1.0996x; 1.0996x over previous
import numpy as np
import jax
import jax.numpy as jnp
from jax.experimental import pallas as pl
from jax.experimental.pallas import tpu as pltpu

_NUM_CLASSES = 80
_CONF = 0.05
_IOU = 0.5
_MPC = 100
_MAXDET = 100
_CAND = 1000
_N = 49104
_NT = 49152
_NB = 1024
_CH = 16
_W = _NT // _CH
_BPC = _W // _NB
_NEG = float("-inf")
_PADV = -1.0e30


def _anchor_table(image_h, image_w):
    aspect_ratios = [0.5, 1.0, 2.0]
    scales = [2 ** x for x in [0.0, 1.0 / 3.0, 2.0 / 3.0]]
    areas = [x ** 2 for x in [32.0, 64.0, 128.0, 256.0, 512.0]]
    all_anchors = []
    for level in range(3, 8):
        stride = 2 ** level
        area = areas[level - 3]
        dims = []
        for ratio in aspect_ratios:
            ah = np.sqrt(area / ratio)
            aw = area / ah
            for s in scales:
                dims.append([aw * s, ah * s])
        dims = np.asarray(dims, dtype=np.float32)
        fh = int(np.ceil(image_h / stride))
        fw = int(np.ceil(image_w / stride))
        rx = (np.arange(fw, dtype=np.float32) + 0.5) * stride
        ry = (np.arange(fh, dtype=np.float32) + 0.5) * stride
        cx, cy = np.meshgrid(rx, ry)
        centers = np.stack([cx, cy], axis=-1).reshape(-1, 1, 2)
        centers = np.tile(centers, (1, 9, 1))
        dims_t = np.tile(dims[None, :, :], (centers.shape[0], 1, 1))
        anchors = np.concatenate([centers, dims_t], axis=-1).reshape(-1, 4)
        all_anchors.append(anchors)
    return np.concatenate(all_anchors, axis=0).astype(np.float32)


def _prep_body(pred_ref, anch_ref, scores_ref, boxes_ref):
    i = pl.program_id(0)
    s = jax.nn.sigmoid(pred_ref[:, 4:84, :])
    col = jax.lax.broadcasted_iota(jnp.int32, s.shape, 2) + i * _NB
    scores_ref[0] = jnp.where(col < _N, s, -1.0)

    acx = anch_ref[:, 0:1, :]
    acy = anch_ref[:, 1:2, :]
    aw = anch_ref[:, 2:3, :]
    ah = anch_ref[:, 3:4, :]
    b0 = pred_ref[:, 0:1, :] * 0.1
    b1 = pred_ref[:, 1:2, :] * 0.1
    b2 = pred_ref[:, 2:3, :] * 0.2
    b3 = pred_ref[:, 3:4, :] * 0.2
    cx = b0 * aw + acx
    cy = b1 * ah + acy
    w = jnp.exp(b2) * aw
    h = jnp.exp(b3) * ah
    x1 = cx - w / 2.0
    y1 = cy - h / 2.0
    x2 = cx + w / 2.0
    y2 = cy + h / 2.0
    area = jnp.maximum(x2 - x1, 0.0) * jnp.maximum(y2 - y1, 0.0)
    zpad = jnp.zeros_like(area)
    boxes_ref[0] = jnp.concatenate(
        [x1, y1, x2, y2, area, zpad, zpad, zpad], axis=1)


def _nms_body(scores_ref, boxes_ref, fo_ref, pcs_ref, px1_ref, py1_ref,
              px2_ref, py2_ref):
    B = scores_ref.shape[1]

    def _civ(c):
        iv = jax.lax.broadcasted_iota(jnp.int32, (B, _NUM_CLASSES, _W), 2)
        return iv + c * _W

    def _count_ge(trial_f):
        def chunk(c, cnt):
            sc = scores_ref[c]
            return cnt + jnp.sum((sc >= trial_f).astype(jnp.int32), axis=2,
                                 keepdims=True)
        return jax.lax.fori_loop(0, _CH, chunk,
                                 jnp.zeros((B, _NUM_CLASSES, 1), jnp.int32))

    def bit_step(j, t):
        trial = t | jax.lax.shift_left(1, 30 - j)
        trial_f = jax.lax.bitcast_convert_type(trial, jnp.float32)
        return jnp.where(_count_ge(trial_f) >= _CAND, trial, t)

    t = jax.lax.fori_loop(0, 31, bit_step, jnp.zeros((B, 80, 1), jnp.int32))
    t_f = jax.lax.bitcast_convert_type(t, jnp.float32)

    def chunk_strict(c, cnt):
        sc = scores_ref[c]
        return cnt + jnp.sum((sc > t_f).astype(jnp.int32), axis=2,
                             keepdims=True)

    n_strict = jax.lax.fori_loop(0, _CH, chunk_strict,
                                 jnp.zeros((B, _NUM_CLASSES, 1), jnp.int32))
    quota = _CAND - n_strict

    def idx_step(j, cc):
        trial = cc | jax.lax.shift_left(1, 16 - j)

        def chunk(c, cnt):
            sc = scores_ref[c]
            hit = (sc == t_f) & (_civ(c) <= trial)
            return cnt + jnp.sum(hit.astype(jnp.int32), axis=2, keepdims=True)

        cnt = jax.lax.fori_loop(0, _CH, chunk,
                                jnp.zeros((B, _NUM_CLASSES, 1), jnp.int32))
        return jnp.where(cnt <= quota, trial, cc)

    cut = jax.lax.fori_loop(0, 17, idx_step, jnp.zeros((B, 80, 1), jnp.int32))

    def chunk_keep(c, carry):
        sc = scores_ref[c]
        keep = (sc > t_f) | ((sc == t_f) & (_civ(c) <= cut))
        scores_ref[c] = jnp.where(keep, sc, _NEG)
        return carry

    jax.lax.fori_loop(0, _CH, chunk_keep, 0)

    pcs_ref[...] = jnp.full(pcs_ref.shape, _PADV, jnp.float32)
    px1_ref[...] = jnp.zeros(px1_ref.shape, jnp.float32)
    py1_ref[...] = jnp.zeros(py1_ref.shape, jnp.float32)
    px2_ref[...] = jnp.zeros(px2_ref.shape, jnp.float32)
    py2_ref[...] = jnp.zeros(py2_ref.shape, jnp.float32)

    slot = jax.lax.broadcasted_iota(jnp.int32, (B, _NUM_CLASSES, 128), 2)

    def nms_step(k, carry):
        def p1(c, mp):
            m2, pick2 = mp
            sc = scores_ref[c]
            cm = jnp.max(sc, axis=2, keepdims=True)
            ci = jnp.min(jnp.where(sc == cm, _civ(c), _NT), axis=2,
                         keepdims=True)
            better = cm > m2
            return (jnp.maximum(m2, cm), jnp.where(better, ci, pick2))

        m, pick = jax.lax.fori_loop(
            0, _CH, p1,
            (jnp.full((B, _NUM_CLASSES, 1), _NEG, jnp.float32),
             jnp.full((B, _NUM_CLASSES, 1), _NT, jnp.int32)))

        def p2(c, acc):
            oh = _civ(c) == pick
            bx = boxes_ref[c]
            out = []
            for r in range(4):
                g = jnp.max(jnp.where(oh, bx[:, r:r + 1, :], _PADV), axis=2,
                            keepdims=True)
                out.append(jnp.maximum(acc[r], g))
            return tuple(out)

        init = tuple(jnp.full((B, _NUM_CLASSES, 1), _PADV, jnp.float32)
                     for _ in range(4))
        p_x1, p_y1, p_x2, p_y2 = jax.lax.fori_loop(0, _CH, p2, init)

        a1 = (jnp.maximum(p_x2 - p_x1, 0.0) *
              jnp.maximum(p_y2 - p_y1, 0.0))

        def p3(c, carry2):
            sc = scores_ref[c]
            bx = boxes_ref[c]
            ix1 = jnp.maximum(p_x1, bx[:, 0:1, :])
            iy1 = jnp.maximum(p_y1, bx[:, 1:2, :])
            ix2 = jnp.minimum(p_x2, bx[:, 2:3, :])
            iy2 = jnp.minimum(p_y2, bx[:, 3:4, :])
            inter = jnp.maximum(ix2 - ix1, 0.0) * jnp.maximum(iy2 - iy1, 0.0)
            iou = inter / jnp.maximum(a1 + bx[:, 4:5, :] - inter, 1e-8)
            kill = (iou > _IOU) | (_civ(c) == pick)
            scores_ref[c] = jnp.where(kill, _NEG, sc)
            return carry2

        jax.lax.fori_loop(0, _CH, p3, 0)

        valid = m >= _CONF
        hit = slot == k
        pcs_ref[...] = jnp.where(hit, jnp.where(valid, m, -1.0), pcs_ref[...])
        px1_ref[...] = jnp.where(hit, jnp.where(valid, p_x1, 0.0), px1_ref[...])
        py1_ref[...] = jnp.where(hit, jnp.where(valid, p_y1, 0.0), py1_ref[...])
        px2_ref[...] = jnp.where(hit, jnp.where(valid, p_x2, 0.0), px2_ref[...])
        py2_ref[...] = jnp.where(hit, jnp.where(valid, p_y2, 0.0), py2_ref[...])
        return carry

    jax.lax.fori_loop(0, _MPC, nms_step, 0)

    ci = jax.lax.broadcasted_iota(jnp.int32, (B, _NUM_CLASSES, 128), 1)
    flatv = ci * 128 + slot
    orow = jax.lax.broadcasted_iota(jnp.int32, (B, 128, 8), 1)

    fo_ref[...] = jnp.zeros(fo_ref.shape, jnp.float32)

    def merge_step(k, nv):
        fs = pcs_ref[...]
        m2 = jnp.max(jnp.max(fs, axis=2, keepdims=True), axis=1, keepdims=True)
        pf = jnp.where(fs == m2, flatv, 2 ** 30)
        pf = jnp.min(jnp.min(pf, axis=2, keepdims=True), axis=1, keepdims=True)
        onehot = flatv == pf
        pcs_ref[...] = jnp.where(onehot, _PADV, fs)

        def gath(r):
            g = jnp.where(onehot, r[...], _PADV)
            return jnp.max(jnp.max(g, axis=2, keepdims=True), axis=1,
                           keepdims=True)

        v = m2 > -0.5
        nv = nv + v.astype(jnp.float32)
        ws = jnp.where(v, m2, 0.0)
        wx1 = jnp.where(v, gath(px1_ref), 0.0)
        wy1 = jnp.where(v, gath(py1_ref), 0.0)
        wx2 = jnp.where(v, gath(px2_ref), 0.0)
        wy2 = jnp.where(v, gath(py2_ref), 0.0)
        wc = jnp.where(v, (pf // 128).astype(jnp.float32), 0.0)
        row = jnp.concatenate([ws, wx1, wy1, wx2, wy2, wc, nv,
                               jnp.zeros_like(ws)], axis=2)
        fo_ref[...] = jnp.where(orow == k, row, fo_ref[...])
        return nv

    jax.lax.fori_loop(0, _MAXDET, merge_step,
                      jnp.zeros((B, 1, 1), jnp.float32))


def kernel(images, predictions):
    B, n, _ = predictions.shape
    del images
    anch = _anchor_table(512.0, 512.0)
    anch = np.pad(anch, ((0, _NT - _N), (0, 4)),
                  constant_values=1.0)
    anch_t = jnp.asarray(anch.T[None], jnp.float32)

    pred = jnp.pad(predictions, ((0, 0), (0, _NT - n), (0, 0)))
    pred_t = jnp.transpose(pred, (0, 2, 1))

    scores, boxes = pl.pallas_call(
        _prep_body,
        grid=(_NT // _NB,),
        in_specs=[
            pl.BlockSpec((B, 84, _NB), lambda i: (0, 0, i)),
            pl.BlockSpec((1, 8, _NB), lambda i: (0, 0, i)),
        ],
        out_specs=[
            pl.BlockSpec((1, B, 80, _NB),
                         lambda i: (i // _BPC, 0, 0, i % _BPC)),
            pl.BlockSpec((1, B, 8, _NB),
                         lambda i: (i // _BPC, 0, 0, i % _BPC)),
        ],
        out_shape=[
            jax.ShapeDtypeStruct((_CH, B, 80, _W), jnp.float32),
            jax.ShapeDtypeStruct((_CH, B, 8, _W), jnp.float32),
        ],
    )(pred_t, anch_t)

    fo = pl.pallas_call(
        _nms_body,
        out_shape=jax.ShapeDtypeStruct((B, 128, 8), jnp.float32),
        scratch_shapes=[
            pltpu.VMEM((B, _NUM_CLASSES, 128), jnp.float32),
            pltpu.VMEM((B, _NUM_CLASSES, 128), jnp.float32),
            pltpu.VMEM((B, _NUM_CLASSES, 128), jnp.float32),
            pltpu.VMEM((B, _NUM_CLASSES, 128), jnp.float32),
            pltpu.VMEM((B, _NUM_CLASSES, 128), jnp.float32),
        ],
    )(scores, boxes)

    out_scores = fo[:, :_MAXDET, 0]
    out_boxes = fo[:, :_MAXDET, 1:5]
    out_classes = fo[:, :_MAXDET, 5]
    n_valid = fo[:, _MAXDET - 1, 6].astype(jnp.int32)
    return out_boxes, out_scores, out_classes, n_valid


# speedup vs baseline: 1.1081x; 1.0077x over previous
import numpy as np
import jax
import jax.numpy as jnp
from jax.experimental import pallas as pl
from jax.experimental.pallas import tpu as pltpu

_NUM_CLASSES = 80
_CONF = 0.05
_IOU = 0.5
_MPC = 100
_MAXDET = 100
_CAND = 1000
_N = 49104
_NT = 49152
_NB = 1024
_CH = 12
_W = _NT // _CH
_BPC = _W // _NB
_NEG = float("-inf")
_PADV = -1.0e30


def _anchor_table(image_h, image_w):
    aspect_ratios = [0.5, 1.0, 2.0]
    scales = [2 ** x for x in [0.0, 1.0 / 3.0, 2.0 / 3.0]]
    areas = [x ** 2 for x in [32.0, 64.0, 128.0, 256.0, 512.0]]
    all_anchors = []
    for level in range(3, 8):
        stride = 2 ** level
        area = areas[level - 3]
        dims = []
        for ratio in aspect_ratios:
            ah = np.sqrt(area / ratio)
            aw = area / ah
            for s in scales:
                dims.append([aw * s, ah * s])
        dims = np.asarray(dims, dtype=np.float32)
        fh = int(np.ceil(image_h / stride))
        fw = int(np.ceil(image_w / stride))
        rx = (np.arange(fw, dtype=np.float32) + 0.5) * stride
        ry = (np.arange(fh, dtype=np.float32) + 0.5) * stride
        cx, cy = np.meshgrid(rx, ry)
        centers = np.stack([cx, cy], axis=-1).reshape(-1, 1, 2)
        centers = np.tile(centers, (1, 9, 1))
        dims_t = np.tile(dims[None, :, :], (centers.shape[0], 1, 1))
        anchors = np.concatenate([centers, dims_t], axis=-1).reshape(-1, 4)
        all_anchors.append(anchors)
    return np.concatenate(all_anchors, axis=0).astype(np.float32)


def _prep_body(pred_ref, anch_ref, scores_ref, boxes_ref):
    i = pl.program_id(0)
    s = jax.nn.sigmoid(pred_ref[:, 4:84, :])
    col = jax.lax.broadcasted_iota(jnp.int32, s.shape, 2) + i * _NB
    scores_ref[0] = jnp.where(col < _N, s, -1.0)

    acx = anch_ref[:, 0:1, :]
    acy = anch_ref[:, 1:2, :]
    aw = anch_ref[:, 2:3, :]
    ah = anch_ref[:, 3:4, :]
    b0 = pred_ref[:, 0:1, :] * 0.1
    b1 = pred_ref[:, 1:2, :] * 0.1
    b2 = pred_ref[:, 2:3, :] * 0.2
    b3 = pred_ref[:, 3:4, :] * 0.2
    cx = b0 * aw + acx
    cy = b1 * ah + acy
    w = jnp.exp(b2) * aw
    h = jnp.exp(b3) * ah
    x1 = cx - w / 2.0
    y1 = cy - h / 2.0
    x2 = cx + w / 2.0
    y2 = cy + h / 2.0
    area = jnp.maximum(x2 - x1, 0.0) * jnp.maximum(y2 - y1, 0.0)
    zpad = jnp.zeros_like(area)
    boxes_ref[0] = jnp.concatenate(
        [x1, y1, x2, y2, area, zpad, zpad, zpad], axis=1)


def _nms_body(scores_ref, boxes_ref, fo_ref, pcs_ref, px1_ref, py1_ref,
              px2_ref, py2_ref):
    B = scores_ref.shape[1]

    def _civ(c):
        iv = jax.lax.broadcasted_iota(jnp.int32, (B, _NUM_CLASSES, _W), 2)
        return iv + c * _W

    def _count_ge(trial_f):
        def chunk(c, cnt):
            sc = scores_ref[c]
            return cnt + jnp.sum((sc >= trial_f).astype(jnp.int32), axis=2,
                                 keepdims=True)
        return jax.lax.fori_loop(0, _CH, chunk,
                                 jnp.zeros((B, _NUM_CLASSES, 1), jnp.int32))

    def bit_step(j, t):
        trial = t | jax.lax.shift_left(1, 30 - j)
        trial_f = jax.lax.bitcast_convert_type(trial, jnp.float32)
        return jnp.where(_count_ge(trial_f) >= _CAND, trial, t)

    t = jax.lax.fori_loop(0, 31, bit_step, jnp.zeros((B, 80, 1), jnp.int32))
    t_f = jax.lax.bitcast_convert_type(t, jnp.float32)

    def chunk_strict(c, cnt):
        sc = scores_ref[c]
        return cnt + jnp.sum((sc > t_f).astype(jnp.int32), axis=2,
                             keepdims=True)

    n_strict = jax.lax.fori_loop(0, _CH, chunk_strict,
                                 jnp.zeros((B, _NUM_CLASSES, 1), jnp.int32))
    quota = _CAND - n_strict

    def idx_step(j, cc):
        trial = cc | jax.lax.shift_left(1, 16 - j)

        def chunk(c, cnt):
            sc = scores_ref[c]
            hit = (sc == t_f) & (_civ(c) <= trial)
            return cnt + jnp.sum(hit.astype(jnp.int32), axis=2, keepdims=True)

        cnt = jax.lax.fori_loop(0, _CH, chunk,
                                jnp.zeros((B, _NUM_CLASSES, 1), jnp.int32))
        return jnp.where(cnt <= quota, trial, cc)

    cut = jax.lax.fori_loop(0, 17, idx_step, jnp.zeros((B, 80, 1), jnp.int32))

    def chunk_keep(c, carry):
        sc = scores_ref[c]
        keep = (sc > t_f) | ((sc == t_f) & (_civ(c) <= cut))
        scores_ref[c] = jnp.where(keep, sc, _NEG)
        return carry

    jax.lax.fori_loop(0, _CH, chunk_keep, 0)

    pcs_ref[...] = jnp.full(pcs_ref.shape, _PADV, jnp.float32)
    px1_ref[...] = jnp.zeros(px1_ref.shape, jnp.float32)
    py1_ref[...] = jnp.zeros(py1_ref.shape, jnp.float32)
    px2_ref[...] = jnp.zeros(px2_ref.shape, jnp.float32)
    py2_ref[...] = jnp.zeros(py2_ref.shape, jnp.float32)

    slot = jax.lax.broadcasted_iota(jnp.int32, (B, _NUM_CLASSES, 128), 2)

    def nms_step(k, carry):
        def p1(c, mp):
            m2, pick2 = mp
            sc = scores_ref[c]
            cm = jnp.max(sc, axis=2, keepdims=True)
            ci = jnp.min(jnp.where(sc == cm, _civ(c), _NT), axis=2,
                         keepdims=True)
            better = cm > m2
            return (jnp.maximum(m2, cm), jnp.where(better, ci, pick2))

        m, pick = jax.lax.fori_loop(
            0, _CH, p1,
            (jnp.full((B, _NUM_CLASSES, 1), _NEG, jnp.float32),
             jnp.full((B, _NUM_CLASSES, 1), _NT, jnp.int32)))

        def p2(c, acc):
            oh = _civ(c) == pick
            bx = boxes_ref[c]
            out = []
            for r in range(4):
                g = jnp.max(jnp.where(oh, bx[:, r:r + 1, :], _PADV), axis=2,
                            keepdims=True)
                out.append(jnp.maximum(acc[r], g))
            return tuple(out)

        init = tuple(jnp.full((B, _NUM_CLASSES, 1), _PADV, jnp.float32)
                     for _ in range(4))
        p_x1, p_y1, p_x2, p_y2 = jax.lax.fori_loop(0, _CH, p2, init)

        a1 = (jnp.maximum(p_x2 - p_x1, 0.0) *
              jnp.maximum(p_y2 - p_y1, 0.0))

        def p3(c, carry2):
            sc = scores_ref[c]
            bx = boxes_ref[c]
            ix1 = jnp.maximum(p_x1, bx[:, 0:1, :])
            iy1 = jnp.maximum(p_y1, bx[:, 1:2, :])
            ix2 = jnp.minimum(p_x2, bx[:, 2:3, :])
            iy2 = jnp.minimum(p_y2, bx[:, 3:4, :])
            inter = jnp.maximum(ix2 - ix1, 0.0) * jnp.maximum(iy2 - iy1, 0.0)
            iou = inter / jnp.maximum(a1 + bx[:, 4:5, :] - inter, 1e-8)
            kill = (iou > _IOU) | (_civ(c) == pick)
            scores_ref[c] = jnp.where(kill, _NEG, sc)
            return carry2

        jax.lax.fori_loop(0, _CH, p3, 0)

        valid = m >= _CONF
        hit = slot == k
        pcs_ref[...] = jnp.where(hit, jnp.where(valid, m, -1.0), pcs_ref[...])
        px1_ref[...] = jnp.where(hit, jnp.where(valid, p_x1, 0.0), px1_ref[...])
        py1_ref[...] = jnp.where(hit, jnp.where(valid, p_y1, 0.0), py1_ref[...])
        px2_ref[...] = jnp.where(hit, jnp.where(valid, p_x2, 0.0), px2_ref[...])
        py2_ref[...] = jnp.where(hit, jnp.where(valid, p_y2, 0.0), py2_ref[...])
        return carry

    jax.lax.fori_loop(0, _MPC, nms_step, 0)

    ci = jax.lax.broadcasted_iota(jnp.int32, (B, _NUM_CLASSES, 128), 1)
    flatv = ci * 128 + slot
    orow = jax.lax.broadcasted_iota(jnp.int32, (B, 128, 8), 1)

    fo_ref[...] = jnp.zeros(fo_ref.shape, jnp.float32)

    def merge_step(k, nv):
        fs = pcs_ref[...]
        m2 = jnp.max(jnp.max(fs, axis=2, keepdims=True), axis=1, keepdims=True)
        pf = jnp.where(fs == m2, flatv, 2 ** 30)
        pf = jnp.min(jnp.min(pf, axis=2, keepdims=True), axis=1, keepdims=True)
        onehot = flatv == pf
        pcs_ref[...] = jnp.where(onehot, _PADV, fs)

        def gath(r):
            g = jnp.where(onehot, r[...], _PADV)
            return jnp.max(jnp.max(g, axis=2, keepdims=True), axis=1,
                           keepdims=True)

        v = m2 > -0.5
        nv = nv + v.astype(jnp.float32)
        ws = jnp.where(v, m2, 0.0)
        wx1 = jnp.where(v, gath(px1_ref), 0.0)
        wy1 = jnp.where(v, gath(py1_ref), 0.0)
        wx2 = jnp.where(v, gath(px2_ref), 0.0)
        wy2 = jnp.where(v, gath(py2_ref), 0.0)
        wc = jnp.where(v, (pf // 128).astype(jnp.float32), 0.0)
        row = jnp.concatenate([ws, wx1, wy1, wx2, wy2, wc, nv,
                               jnp.zeros_like(ws)], axis=2)
        fo_ref[...] = jnp.where(orow == k, row, fo_ref[...])
        return nv

    jax.lax.fori_loop(0, _MAXDET, merge_step,
                      jnp.zeros((B, 1, 1), jnp.float32))


def kernel(images, predictions):
    B, n, _ = predictions.shape
    del images
    anch = _anchor_table(512.0, 512.0)
    anch = np.pad(anch, ((0, _NT - _N), (0, 4)),
                  constant_values=1.0)
    anch_t = jnp.asarray(anch.T[None], jnp.float32)

    pred = jnp.pad(predictions, ((0, 0), (0, _NT - n), (0, 0)))
    pred_t = jnp.transpose(pred, (0, 2, 1))

    scores, boxes = pl.pallas_call(
        _prep_body,
        grid=(_NT // _NB,),
        in_specs=[
            pl.BlockSpec((B, 84, _NB), lambda i: (0, 0, i)),
            pl.BlockSpec((1, 8, _NB), lambda i: (0, 0, i)),
        ],
        out_specs=[
            pl.BlockSpec((1, B, 80, _NB),
                         lambda i: (i // _BPC, 0, 0, i % _BPC)),
            pl.BlockSpec((1, B, 8, _NB),
                         lambda i: (i // _BPC, 0, 0, i % _BPC)),
        ],
        out_shape=[
            jax.ShapeDtypeStruct((_CH, B, 80, _W), jnp.float32),
            jax.ShapeDtypeStruct((_CH, B, 8, _W), jnp.float32),
        ],
    )(pred_t, anch_t)

    fo = pl.pallas_call(
        _nms_body,
        out_shape=jax.ShapeDtypeStruct((B, 128, 8), jnp.float32),
        scratch_shapes=[
            pltpu.VMEM((B, _NUM_CLASSES, 128), jnp.float32),
            pltpu.VMEM((B, _NUM_CLASSES, 128), jnp.float32),
            pltpu.VMEM((B, _NUM_CLASSES, 128), jnp.float32),
            pltpu.VMEM((B, _NUM_CLASSES, 128), jnp.float32),
            pltpu.VMEM((B, _NUM_CLASSES, 128), jnp.float32),
        ],
    )(scores, boxes)

    out_scores = fo[:, :_MAXDET, 0]
    out_boxes = fo[:, :_MAXDET, 1:5]
    out_classes = fo[:, :_MAXDET, 5]
    n_valid = fo[:, _MAXDET - 1, 6].astype(jnp.int32)
    return out_boxes, out_scores, out_classes, n_valid


# speedup vs baseline: 1.1112x; 1.0028x over previous
import numpy as np
import jax
import jax.numpy as jnp
from jax.experimental import pallas as pl
from jax.experimental.pallas import tpu as pltpu

_NUM_CLASSES = 80
_CONF = 0.05
_IOU = 0.5
_MPC = 100
_MAXDET = 100
_CAND = 1000
_N = 49104
_NT = 49152
_NB = 1024
_CH = 8
_W = _NT // _CH
_BPC = _W // _NB
_NEG = float("-inf")
_PADV = -1.0e30


def _anchor_table(image_h, image_w):
    aspect_ratios = [0.5, 1.0, 2.0]
    scales = [2 ** x for x in [0.0, 1.0 / 3.0, 2.0 / 3.0]]
    areas = [x ** 2 for x in [32.0, 64.0, 128.0, 256.0, 512.0]]
    all_anchors = []
    for level in range(3, 8):
        stride = 2 ** level
        area = areas[level - 3]
        dims = []
        for ratio in aspect_ratios:
            ah = np.sqrt(area / ratio)
            aw = area / ah
            for s in scales:
                dims.append([aw * s, ah * s])
        dims = np.asarray(dims, dtype=np.float32)
        fh = int(np.ceil(image_h / stride))
        fw = int(np.ceil(image_w / stride))
        rx = (np.arange(fw, dtype=np.float32) + 0.5) * stride
        ry = (np.arange(fh, dtype=np.float32) + 0.5) * stride
        cx, cy = np.meshgrid(rx, ry)
        centers = np.stack([cx, cy], axis=-1).reshape(-1, 1, 2)
        centers = np.tile(centers, (1, 9, 1))
        dims_t = np.tile(dims[None, :, :], (centers.shape[0], 1, 1))
        anchors = np.concatenate([centers, dims_t], axis=-1).reshape(-1, 4)
        all_anchors.append(anchors)
    return np.concatenate(all_anchors, axis=0).astype(np.float32)


def _prep_body(pred_ref, anch_ref, scores_ref, boxes_ref):
    i = pl.program_id(0)
    s = jax.nn.sigmoid(pred_ref[:, 4:84, :])
    col = jax.lax.broadcasted_iota(jnp.int32, s.shape, 2) + i * _NB
    scores_ref[0] = jnp.where(col < _N, s, -1.0)

    acx = anch_ref[:, 0:1, :]
    acy = anch_ref[:, 1:2, :]
    aw = anch_ref[:, 2:3, :]
    ah = anch_ref[:, 3:4, :]
    b0 = pred_ref[:, 0:1, :] * 0.1
    b1 = pred_ref[:, 1:2, :] * 0.1
    b2 = pred_ref[:, 2:3, :] * 0.2
    b3 = pred_ref[:, 3:4, :] * 0.2
    cx = b0 * aw + acx
    cy = b1 * ah + acy
    w = jnp.exp(b2) * aw
    h = jnp.exp(b3) * ah
    x1 = cx - w / 2.0
    y1 = cy - h / 2.0
    x2 = cx + w / 2.0
    y2 = cy + h / 2.0
    area = jnp.maximum(x2 - x1, 0.0) * jnp.maximum(y2 - y1, 0.0)
    zpad = jnp.zeros_like(area)
    boxes_ref[0] = jnp.concatenate(
        [x1, y1, x2, y2, area, zpad, zpad, zpad], axis=1)


def _nms_body(scores_ref, boxes_ref, fo_ref, pcs_ref, px1_ref, py1_ref,
              px2_ref, py2_ref):
    B = scores_ref.shape[1]

    def _civ(c):
        iv = jax.lax.broadcasted_iota(jnp.int32, (B, _NUM_CLASSES, _W), 2)
        return iv + c * _W

    def _count_ge(trial_f):
        def chunk(c, cnt):
            sc = scores_ref[c]
            return cnt + jnp.sum((sc >= trial_f).astype(jnp.int32), axis=2,
                                 keepdims=True)
        return jax.lax.fori_loop(0, _CH, chunk,
                                 jnp.zeros((B, _NUM_CLASSES, 1), jnp.int32))

    def bit_step(j, t):
        trial = t | jax.lax.shift_left(1, 30 - j)
        trial_f = jax.lax.bitcast_convert_type(trial, jnp.float32)
        return jnp.where(_count_ge(trial_f) >= _CAND, trial, t)

    t = jax.lax.fori_loop(0, 31, bit_step, jnp.zeros((B, 80, 1), jnp.int32))
    t_f = jax.lax.bitcast_convert_type(t, jnp.float32)

    def chunk_strict(c, cnt):
        sc = scores_ref[c]
        return cnt + jnp.sum((sc > t_f).astype(jnp.int32), axis=2,
                             keepdims=True)

    n_strict = jax.lax.fori_loop(0, _CH, chunk_strict,
                                 jnp.zeros((B, _NUM_CLASSES, 1), jnp.int32))
    quota = _CAND - n_strict

    def idx_step(j, cc):
        trial = cc | jax.lax.shift_left(1, 16 - j)

        def chunk(c, cnt):
            sc = scores_ref[c]
            hit = (sc == t_f) & (_civ(c) <= trial)
            return cnt + jnp.sum(hit.astype(jnp.int32), axis=2, keepdims=True)

        cnt = jax.lax.fori_loop(0, _CH, chunk,
                                jnp.zeros((B, _NUM_CLASSES, 1), jnp.int32))
        return jnp.where(cnt <= quota, trial, cc)

    cut = jax.lax.fori_loop(0, 17, idx_step, jnp.zeros((B, 80, 1), jnp.int32))

    def chunk_keep(c, carry):
        sc = scores_ref[c]
        keep = (sc > t_f) | ((sc == t_f) & (_civ(c) <= cut))
        scores_ref[c] = jnp.where(keep, sc, _NEG)
        return carry

    jax.lax.fori_loop(0, _CH, chunk_keep, 0)

    pcs_ref[...] = jnp.full(pcs_ref.shape, _PADV, jnp.float32)
    px1_ref[...] = jnp.zeros(px1_ref.shape, jnp.float32)
    py1_ref[...] = jnp.zeros(py1_ref.shape, jnp.float32)
    px2_ref[...] = jnp.zeros(px2_ref.shape, jnp.float32)
    py2_ref[...] = jnp.zeros(py2_ref.shape, jnp.float32)

    slot = jax.lax.broadcasted_iota(jnp.int32, (B, _NUM_CLASSES, 128), 2)

    def nms_step(k, carry):
        def p1(c, mp):
            m2, pick2 = mp
            sc = scores_ref[c]
            cm = jnp.max(sc, axis=2, keepdims=True)
            ci = jnp.min(jnp.where(sc == cm, _civ(c), _NT), axis=2,
                         keepdims=True)
            better = cm > m2
            return (jnp.maximum(m2, cm), jnp.where(better, ci, pick2))

        m, pick = jax.lax.fori_loop(
            0, _CH, p1,
            (jnp.full((B, _NUM_CLASSES, 1), _NEG, jnp.float32),
             jnp.full((B, _NUM_CLASSES, 1), _NT, jnp.int32)))

        def p2(c, acc):
            oh = _civ(c) == pick
            bx = boxes_ref[c]
            out = []
            for r in range(4):
                g = jnp.max(jnp.where(oh, bx[:, r:r + 1, :], _PADV), axis=2,
                            keepdims=True)
                out.append(jnp.maximum(acc[r], g))
            return tuple(out)

        init = tuple(jnp.full((B, _NUM_CLASSES, 1), _PADV, jnp.float32)
                     for _ in range(4))
        p_x1, p_y1, p_x2, p_y2 = jax.lax.fori_loop(0, _CH, p2, init)

        a1 = (jnp.maximum(p_x2 - p_x1, 0.0) *
              jnp.maximum(p_y2 - p_y1, 0.0))

        def p3(c, carry2):
            sc = scores_ref[c]
            bx = boxes_ref[c]
            ix1 = jnp.maximum(p_x1, bx[:, 0:1, :])
            iy1 = jnp.maximum(p_y1, bx[:, 1:2, :])
            ix2 = jnp.minimum(p_x2, bx[:, 2:3, :])
            iy2 = jnp.minimum(p_y2, bx[:, 3:4, :])
            inter = jnp.maximum(ix2 - ix1, 0.0) * jnp.maximum(iy2 - iy1, 0.0)
            iou = inter / jnp.maximum(a1 + bx[:, 4:5, :] - inter, 1e-8)
            kill = (iou > _IOU) | (_civ(c) == pick)
            scores_ref[c] = jnp.where(kill, _NEG, sc)
            return carry2

        jax.lax.fori_loop(0, _CH, p3, 0)

        valid = m >= _CONF
        hit = slot == k
        pcs_ref[...] = jnp.where(hit, jnp.where(valid, m, -1.0), pcs_ref[...])
        px1_ref[...] = jnp.where(hit, jnp.where(valid, p_x1, 0.0), px1_ref[...])
        py1_ref[...] = jnp.where(hit, jnp.where(valid, p_y1, 0.0), py1_ref[...])
        px2_ref[...] = jnp.where(hit, jnp.where(valid, p_x2, 0.0), px2_ref[...])
        py2_ref[...] = jnp.where(hit, jnp.where(valid, p_y2, 0.0), py2_ref[...])
        return carry

    jax.lax.fori_loop(0, _MPC, nms_step, 0)

    ci = jax.lax.broadcasted_iota(jnp.int32, (B, _NUM_CLASSES, 128), 1)
    flatv = ci * 128 + slot
    orow = jax.lax.broadcasted_iota(jnp.int32, (B, 128, 8), 1)

    fo_ref[...] = jnp.zeros(fo_ref.shape, jnp.float32)

    def merge_step(k, nv):
        fs = pcs_ref[...]
        m2 = jnp.max(jnp.max(fs, axis=2, keepdims=True), axis=1, keepdims=True)
        pf = jnp.where(fs == m2, flatv, 2 ** 30)
        pf = jnp.min(jnp.min(pf, axis=2, keepdims=True), axis=1, keepdims=True)
        onehot = flatv == pf
        pcs_ref[...] = jnp.where(onehot, _PADV, fs)

        def gath(r):
            g = jnp.where(onehot, r[...], _PADV)
            return jnp.max(jnp.max(g, axis=2, keepdims=True), axis=1,
                           keepdims=True)

        v = m2 > -0.5
        nv = nv + v.astype(jnp.float32)
        ws = jnp.where(v, m2, 0.0)
        wx1 = jnp.where(v, gath(px1_ref), 0.0)
        wy1 = jnp.where(v, gath(py1_ref), 0.0)
        wx2 = jnp.where(v, gath(px2_ref), 0.0)
        wy2 = jnp.where(v, gath(py2_ref), 0.0)
        wc = jnp.where(v, (pf // 128).astype(jnp.float32), 0.0)
        row = jnp.concatenate([ws, wx1, wy1, wx2, wy2, wc, nv,
                               jnp.zeros_like(ws)], axis=2)
        fo_ref[...] = jnp.where(orow == k, row, fo_ref[...])
        return nv

    jax.lax.fori_loop(0, _MAXDET, merge_step,
                      jnp.zeros((B, 1, 1), jnp.float32))


def kernel(images, predictions):
    B, n, _ = predictions.shape
    del images
    anch = _anchor_table(512.0, 512.0)
    anch = np.pad(anch, ((0, _NT - _N), (0, 4)),
                  constant_values=1.0)
    anch_t = jnp.asarray(anch.T[None], jnp.float32)

    pred = jnp.pad(predictions, ((0, 0), (0, _NT - n), (0, 0)))
    pred_t = jnp.transpose(pred, (0, 2, 1))

    scores, boxes = pl.pallas_call(
        _prep_body,
        grid=(_NT // _NB,),
        in_specs=[
            pl.BlockSpec((B, 84, _NB), lambda i: (0, 0, i)),
            pl.BlockSpec((1, 8, _NB), lambda i: (0, 0, i)),
        ],
        out_specs=[
            pl.BlockSpec((1, B, 80, _NB),
                         lambda i: (i // _BPC, 0, 0, i % _BPC)),
            pl.BlockSpec((1, B, 8, _NB),
                         lambda i: (i // _BPC, 0, 0, i % _BPC)),
        ],
        out_shape=[
            jax.ShapeDtypeStruct((_CH, B, 80, _W), jnp.float32),
            jax.ShapeDtypeStruct((_CH, B, 8, _W), jnp.float32),
        ],
    )(pred_t, anch_t)

    fo = pl.pallas_call(
        _nms_body,
        out_shape=jax.ShapeDtypeStruct((B, 128, 8), jnp.float32),
        scratch_shapes=[
            pltpu.VMEM((B, _NUM_CLASSES, 128), jnp.float32),
            pltpu.VMEM((B, _NUM_CLASSES, 128), jnp.float32),
            pltpu.VMEM((B, _NUM_CLASSES, 128), jnp.float32),
            pltpu.VMEM((B, _NUM_CLASSES, 128), jnp.float32),
            pltpu.VMEM((B, _NUM_CLASSES, 128), jnp.float32),
        ],
    )(scores, boxes)

    out_scores = fo[:, :_MAXDET, 0]
    out_boxes = fo[:, :_MAXDET, 1:5]
    out_classes = fo[:, :_MAXDET, 5]
    n_valid = fo[:, _MAXDET - 1, 6].astype(jnp.int32)
    return out_boxes, out_scores, out_classes, n_valid
